# baseline (speedup 1.0000x reference)
import jax
import jax.numpy as jnp
from jax import lax
from jax.experimental import pallas as pl
from jax.experimental.pallas import tpu as pltpu
from jax.experimental.pallas import tpu_sc as plsc

_B = 128
_N = 32768
_K = 6
_EPS = 1e-05
_L = 16
_NC = 2
_NS = 16
_NW = _NC * _NS
_RPW = _B // _NW
_UNROLL = 8
_STEPS = _N // _L


def _lane_gather(src, idx):
    dnums = lax.GatherDimensionNumbers(
        offset_dims=(),
        collapsed_slice_dims=(0,),
        start_index_map=(0,),
    )
    return lax.gather(
        src, idx[:, None], dnums, (1,),
        indices_are_sorted=False, unique_indices=False,
        mode=lax.GatherScatterMode.PROMISE_IN_BOUNDS)


def _sc_body(x_hbm, out_hbm, rb0, rb1, cand, o_buf, sem0, sem1):
    wid = lax.axis_index("s") * _NC + lax.axis_index("c")
    row0 = wid * _RPW

    iota = lax.broadcasted_iota(jnp.int32, (_L,), 0)
    pos_inf = jnp.full((_L,), jnp.inf, dtype=jnp.float32)
    acc_c = jnp.zeros((_L,), jnp.float32)

    bufs = (rb0, rb1)
    sems = (sem0, sem1)
    dma = pltpu.async_copy(x_hbm.at[row0], rb0, sem0)

    for r in range(_RPW):
        rb = bufs[r % 2]
        dma.wait()
        if r + 1 < _RPW:
            dma = pltpu.async_copy(
                x_hbm.at[row0 + r + 1], bufs[(r + 1) % 2], sems[(r + 1) % 2])

        def p1(i, carry):
            pm0, pm1, acc0, acc1 = carry
            base = i * (_L * _UNROLL)
            for j in range(_UNROLL):
                v = rb[pl.ds(base + j * _L, _L)]
                if j % 2 == 0:
                    pm0 = jnp.minimum(pm0, v)
                    acc0 = acc0 + jnp.exp(v)
                else:
                    pm1 = jnp.minimum(pm1, v)
                    acc1 = acc1 + jnp.exp(v)
            return pm0, pm1, acc0, acc1

        pm0, pm1, acc0, acc1 = lax.fori_loop(
            0, _STEPS // _UNROLL, p1,
            (pos_inf, pos_inf, jnp.zeros((_L,), jnp.float32),
             jnp.zeros((_L,), jnp.float32)))
        pm = jnp.minimum(pm0, pm1)
        acc = acc0 + acc1

        s_total = acc
        for sh in (8, 4, 2, 1):
            s_total = s_total + _lane_gather(s_total, iota ^ sh)

        t6 = _lane_gather(jnp.sort(pm), jnp.full((_L,), _K - 1, jnp.int32))

        def p2(i, off):
            v = rb[pl.ds(i * _L, _L)]
            m = v <= t6
            pc = plsc.all_reduce_population_count(m)
            plsc.store_scatter(cand, [off + iota], v)
            return off + jnp.where(pc > 0, _L, 0)

        off = lax.fori_loop(0, _STEPS, p2, jnp.zeros((_L,), jnp.int32))
        ntrips = off[0] // _L

        def sel(i, carry):
            t0, t1, t2, t3, t4, t5 = carry
            c = cand[pl.ds(i * _L, _L)]
            n0 = jnp.minimum(t0, c); c = jnp.maximum(t0, c); t0 = n0
            n1 = jnp.minimum(t1, c); c = jnp.maximum(t1, c); t1 = n1
            n2 = jnp.minimum(t2, c); c = jnp.maximum(t2, c); t2 = n2
            n3 = jnp.minimum(t3, c); c = jnp.maximum(t3, c); t3 = n3
            n4 = jnp.minimum(t4, c); c = jnp.maximum(t4, c); t4 = n4
            t5 = jnp.minimum(t5, c)
            return t0, t1, t2, t3, t4, t5

        t0, t1, t2, t3, t4, t5 = lax.fori_loop(
            0, ntrips, sel, (pos_inf,) * 6)

        s = jnp.sort(t0)
        for t in (t1, t2, t3, t4, t5):
            s = jnp.sort(jnp.minimum(s, jnp.flip(jnp.sort(t))))

        sm = jnp.exp(s) / s_total
        t = _EPS - sm
        contrib = t * t * 0.5 - t
        acc_c = acc_c + jnp.where(iota < _K, contrib, 0.0)

    o_buf[...] = acc_c
    pltpu.sync_copy(o_buf, out_hbm.at[wid])


_sc_call = pl.kernel(
    _sc_body,
    out_type=jax.ShapeDtypeStruct((_NW, _L), jnp.float32),
    mesh=plsc.VectorSubcoreMesh(core_axis_name="c", subcore_axis_name="s"),
    scratch_types=[
        pltpu.VMEM((_N,), jnp.float32),
        pltpu.VMEM((_N,), jnp.float32),
        pltpu.VMEM((_N,), jnp.float32),
        pltpu.VMEM((_L,), jnp.float32),
        pltpu.SemaphoreType.DMA,
        pltpu.SemaphoreType.DMA,
    ],
    compiler_params=pltpu.CompilerParams(needs_layout_passes=False),
)


def kernel(x):
    part = _sc_call(x)
    return jnp.sum(part) * (1.0 / (_K * _B))


# speedup vs baseline: 1.3243x; 1.3243x over previous
import jax
import jax.numpy as jnp
from jax import lax
from jax.experimental import pallas as pl
from jax.experimental.pallas import tpu as pltpu
from jax.experimental.pallas import tpu_sc as plsc

_B = 128
_N = 32768
_K = 6
_EPS = 1e-05
_L = 16
_NC = 2
_NS = 16
_NW = _NC * _NS
_RPW = _B // _NW
_UNROLL = 8
_STEPS = _N // _L


def _lane_gather(src, idx):
    dnums = lax.GatherDimensionNumbers(
        offset_dims=(),
        collapsed_slice_dims=(0,),
        start_index_map=(0,),
    )
    return lax.gather(
        src, idx[:, None], dnums, (1,),
        indices_are_sorted=False, unique_indices=False,
        mode=lax.GatherScatterMode.PROMISE_IN_BOUNDS)


def _sc_body(x_hbm, out_hbm, rb0, rb1, cand, o_buf, sem0, sem1):
    wid = lax.axis_index("s") * _NC + lax.axis_index("c")
    row0 = wid * _RPW

    iota = lax.broadcasted_iota(jnp.int32, (_L,), 0)
    pos_inf = jnp.full((_L,), jnp.inf, dtype=jnp.float32)
    acc_c = jnp.zeros((_L,), jnp.float32)

    bufs = (rb0, rb1)
    sems = (sem0, sem1)
    dma = pltpu.async_copy(x_hbm.at[row0], rb0, sem0)

    for r in range(_RPW):
        rb = bufs[r % 2]
        dma.wait()
        if r + 1 < _RPW:
            dma = pltpu.async_copy(
                x_hbm.at[row0 + r + 1], bufs[(r + 1) % 2], sems[(r + 1) % 2])

        def p1(i, carry):
            pm0, pm1, acc0, acc1 = carry
            base = i * (_L * _UNROLL)
            for j in range(_UNROLL):
                v = rb[pl.ds(base + j * _L, _L)]
                if j % 2 == 0:
                    pm0 = jnp.minimum(pm0, v)
                    acc0 = acc0 + jnp.exp(v)
                else:
                    pm1 = jnp.minimum(pm1, v)
                    acc1 = acc1 + jnp.exp(v)
            return pm0, pm1, acc0, acc1

        pm0, pm1, acc0, acc1 = lax.fori_loop(
            0, _STEPS // _UNROLL, p1,
            (pos_inf, pos_inf, jnp.zeros((_L,), jnp.float32),
             jnp.zeros((_L,), jnp.float32)))
        pm = jnp.minimum(pm0, pm1)
        acc = acc0 + acc1

        s_total = acc
        for sh in (8, 4, 2, 1):
            s_total = s_total + _lane_gather(s_total, iota ^ sh)

        t6 = _lane_gather(jnp.sort(pm), jnp.full((_L,), _K - 1, jnp.int32))

        def p2(i, off):
            base = i * (_L * _UNROLL)
            for j in range(_UNROLL // 2):
                va = rb[pl.ds(base + (2 * j) * _L, _L)]
                vb = rb[pl.ds(base + (2 * j + 1) * _L, _L)]
                m = jnp.minimum(va, vb) <= t6
                pc = plsc.all_reduce_population_count(m)
                plsc.store_scatter(cand, [off + iota], va)
                plsc.store_scatter(cand, [off + iota + _L], vb)
                off = off + jnp.where(pc > 0, 2 * _L, 0)
            return off

        off = lax.fori_loop(0, _STEPS // _UNROLL, p2,
                            jnp.zeros((_L,), jnp.int32))
        ntrips = off[0] // _L

        def sel(i, carry):
            t0, t1, t2, t3, t4, t5 = carry
            c = cand[pl.ds(i * _L, _L)]
            n0 = jnp.minimum(t0, c); c = jnp.maximum(t0, c); t0 = n0
            n1 = jnp.minimum(t1, c); c = jnp.maximum(t1, c); t1 = n1
            n2 = jnp.minimum(t2, c); c = jnp.maximum(t2, c); t2 = n2
            n3 = jnp.minimum(t3, c); c = jnp.maximum(t3, c); t3 = n3
            n4 = jnp.minimum(t4, c); c = jnp.maximum(t4, c); t4 = n4
            t5 = jnp.minimum(t5, c)
            return t0, t1, t2, t3, t4, t5

        t0, t1, t2, t3, t4, t5 = lax.fori_loop(
            0, ntrips, sel, (pos_inf,) * 6)

        s = jnp.sort(t0)
        for t in (t1, t2, t3, t4, t5):
            s = jnp.sort(jnp.minimum(s, jnp.flip(jnp.sort(t))))

        sm = jnp.exp(s) / s_total
        t = _EPS - sm
        contrib = t * t * 0.5 - t
        acc_c = acc_c + jnp.where(iota < _K, contrib, 0.0)

    o_buf[...] = acc_c
    pltpu.sync_copy(o_buf, out_hbm.at[wid])


_sc_call = pl.kernel(
    _sc_body,
    out_type=jax.ShapeDtypeStruct((_NW, _L), jnp.float32),
    mesh=plsc.VectorSubcoreMesh(core_axis_name="c", subcore_axis_name="s"),
    scratch_types=[
        pltpu.VMEM((_N,), jnp.float32),
        pltpu.VMEM((_N,), jnp.float32),
        pltpu.VMEM((_N,), jnp.float32),
        pltpu.VMEM((_L,), jnp.float32),
        pltpu.SemaphoreType.DMA,
        pltpu.SemaphoreType.DMA,
    ],
    compiler_params=pltpu.CompilerParams(needs_layout_passes=False),
)


def kernel(x):
    part = _sc_call(x)
    return jnp.sum(part) * (1.0 / (_K * _B))


# speedup vs baseline: 1.7523x; 1.3232x over previous
import jax
import jax.numpy as jnp
from jax import lax
from jax.experimental import pallas as pl
from jax.experimental.pallas import tpu as pltpu
from jax.experimental.pallas import tpu_sc as plsc

_B = 128
_N = 32768
_K = 6
_EPS = 1e-05
_L = 16
_NC = 2
_NS = 16
_NW = _NC * _NS
_RPW = _B // _NW
_UNROLL = 8
_G = 4
_STEPS = _N // _L
_SEL_UNROLL = 4


def _lane_gather(src, idx):
    dnums = lax.GatherDimensionNumbers(
        offset_dims=(),
        collapsed_slice_dims=(0,),
        start_index_map=(0,),
    )
    return lax.gather(
        src, idx[:, None], dnums, (1,),
        indices_are_sorted=False, unique_indices=False,
        mode=lax.GatherScatterMode.PROMISE_IN_BOUNDS)


def _sc_body(x_hbm, out_hbm, rb0, rb1, cand, o_buf, sem0, sem1):
    wid = lax.axis_index("s") * _NC + lax.axis_index("c")
    row0 = wid * _RPW

    iota = lax.broadcasted_iota(jnp.int32, (_L,), 0)
    iotas = [iota + j * _L for j in range(_G)]
    lane5 = jnp.full((_L,), _K - 1, jnp.int32)
    pos_inf = jnp.full((_L,), jnp.inf, dtype=jnp.float32)

    bufs = (rb0, rb1)
    sems = (sem0, sem1)
    dma = pltpu.async_copy(x_hbm.at[row0], rb0, sem0)

    for r in range(_RPW):
        rb = bufs[r % 2]
        dma.wait()
        if r + 1 < _RPW:
            dma = pltpu.async_copy(
                x_hbm.at[row0 + r + 1], bufs[(r + 1) % 2], sems[(r + 1) % 2])

        def p1(i, carry):
            pm0, pm1, pm2, pm3, tau, off = carry
            base = i * (_L * _UNROLL)
            pms = [pm0, pm1, pm2, pm3]
            for g in range(_UNROLL // _G):
                vs = [rb[pl.ds(base + (g * _G + j) * _L, _L)]
                      for j in range(_G)]
                mn01 = jnp.minimum(vs[0], vs[1])
                mn23 = jnp.minimum(vs[2], vs[3])
                mn = jnp.minimum(mn01, mn23)
                pms[2 * g] = jnp.minimum(pms[2 * g], mn01)
                pms[2 * g + 1] = jnp.minimum(pms[2 * g + 1], mn23)
                pc = plsc.all_reduce_population_count(mn <= tau)
                for j in range(_G):
                    plsc.store_scatter(cand, [off + iotas[j]], vs[j])
                off = off + jnp.where(pc > 0, _G * _L, 0)
            pmall = jnp.minimum(jnp.minimum(pms[0], pms[1]),
                                jnp.minimum(pms[2], pms[3]))
            tau = _lane_gather(jnp.sort(pmall), lane5)
            return pms[0], pms[1], pms[2], pms[3], tau, off

        pm0, pm1, pm2, pm3, tau, off = lax.fori_loop(
            0, _STEPS // _UNROLL, p1,
            (pos_inf, pos_inf, pos_inf, pos_inf, pos_inf,
             jnp.zeros((_L,), jnp.int32)))
        ntrips = off[0] // (_SEL_UNROLL * _L)

        def sel(i, carry):
            ts = list(carry)
            base = i * (_SEL_UNROLL * _L)
            for j in range(_SEL_UNROLL):
                c = cand[pl.ds(base + j * _L, _L)]
                for q in range(_K - 1):
                    n = jnp.minimum(ts[q], c)
                    c = jnp.maximum(ts[q], c)
                    ts[q] = n
                ts[_K - 1] = jnp.minimum(ts[_K - 1], c)
            return tuple(ts)

        tsel = lax.fori_loop(0, ntrips, sel, (pos_inf,) * _K)

        s = jnp.sort(tsel[0])
        for t in tsel[1:]:
            s = jnp.sort(jnp.minimum(s, jnp.flip(jnp.sort(t))))

        o_buf[...] = s
        pltpu.sync_copy(o_buf, out_hbm.at[row0 + r])


_sc_call = pl.kernel(
    _sc_body,
    out_type=jax.ShapeDtypeStruct((_B, _L), jnp.float32),
    mesh=plsc.VectorSubcoreMesh(core_axis_name="c", subcore_axis_name="s"),
    scratch_types=[
        pltpu.VMEM((_N,), jnp.float32),
        pltpu.VMEM((_N,), jnp.float32),
        pltpu.VMEM((_N,), jnp.float32),
        pltpu.VMEM((_L,), jnp.float32),
        pltpu.SemaphoreType.DMA,
        pltpu.SemaphoreType.DMA,
    ],
    compiler_params=pltpu.CompilerParams(needs_layout_passes=False),
)

_COLS = 1024


def _tc_rowsum_body(x_ref, o_ref):
    i = pl.program_id(0)
    part = jnp.sum(jnp.exp(x_ref[...]), axis=1, keepdims=True)

    @pl.when(i == 0)
    def _init():
        o_ref[...] = part

    @pl.when(i > 0)
    def _acc():
        o_ref[...] += part


_tc_rowsum = pl.pallas_call(
    _tc_rowsum_body,
    grid=(_N // _COLS,),
    in_specs=[pl.BlockSpec((_B, _COLS), lambda i: (0, i))],
    out_specs=pl.BlockSpec((_B, 1), lambda i: (0, 0)),
    out_shape=jax.ShapeDtypeStruct((_B, 1), jnp.float32),
)


def _tc_comb_body(v_ref, s_ref, o_ref):
    sm = jnp.exp(v_ref[...]) / s_ref[...]
    t = _EPS - sm
    contrib = t * t * 0.5 - t
    keep = lax.broadcasted_iota(jnp.int32, (_B, _L), 1) < _K
    o_ref[0, 0] = jnp.sum(jnp.where(keep, contrib, 0.0)) * (1.0 / (_K * _B))


_tc_comb = pl.pallas_call(
    _tc_comb_body,
    out_shape=jax.ShapeDtypeStruct((1, 1), jnp.float32),
    out_specs=pl.BlockSpec(memory_space=pltpu.SMEM),
)


def kernel(x):
    v6 = _sc_call(x)
    s = _tc_rowsum(x)
    return _tc_comb(v6, s)[0, 0]


# speedup vs baseline: 1.7973x; 1.0257x over previous
import jax
import jax.numpy as jnp
from jax import lax
from jax.experimental import pallas as pl
from jax.experimental.pallas import tpu as pltpu
from jax.experimental.pallas import tpu_sc as plsc

_B = 128
_N = 32768
_K = 6
_EPS = 1e-05
_L = 16
_NC = 2
_NS = 16
_NW = _NC * _NS
_RPW = _B // _NW
_UNROLL = 16
_G = 4
_STEPS = _N // _L
_NGRP = _STEPS // _G


def _lane_gather(src, idx):
    dnums = lax.GatherDimensionNumbers(
        offset_dims=(),
        collapsed_slice_dims=(0,),
        start_index_map=(0,),
    )
    return lax.gather(
        src, idx[:, None], dnums, (1,),
        indices_are_sorted=False, unique_indices=False,
        mode=lax.GatherScatterMode.PROMISE_IN_BOUNDS)


def _sc_body(x_hbm, out_hbm, rb0, rb1, gids, o_buf, sem0, sem1):
    wid = lax.axis_index("s") * _NC + lax.axis_index("c")
    row0 = wid * _RPW

    iota = lax.broadcasted_iota(jnp.int32, (_L,), 0)
    lane5 = jnp.full((_L,), _K - 1, jnp.int32)
    pos_inf = jnp.full((_L,), jnp.inf, dtype=jnp.float32)

    bufs = (rb0, rb1)
    sems = (sem0, sem1)
    dma = pltpu.async_copy(x_hbm.at[row0], rb0, sem0)

    for r in range(_RPW):
        rb = bufs[r % 2]
        dma.wait()
        if r + 1 < _RPW:
            dma = pltpu.async_copy(
                x_hbm.at[row0 + r + 1], bufs[(r + 1) % 2], sems[(r + 1) % 2])

        def p1(i, carry):
            pm0, pm1, pm2, pm3, tau, tau_n, off = carry
            pmall = jnp.minimum(jnp.minimum(pm0, pm1),
                                jnp.minimum(pm2, pm3))
            tau_nn = _lane_gather(jnp.sort(pmall), lane5)
            pms = [pm0, pm1, pm2, pm3]
            base = i * (_L * _UNROLL)
            gid0 = i * (_UNROLL // _G)
            for g in range(_UNROLL // _G):
                vs = [rb[pl.ds(base + (g * _G + j) * _L, _L)]
                      for j in range(_G)]
                mn01 = jnp.minimum(vs[0], vs[1])
                mn23 = jnp.minimum(vs[2], vs[3])
                mn = jnp.minimum(mn01, mn23)
                pms[g] = jnp.minimum(pms[g], mn)
                pc = plsc.all_reduce_population_count(mn <= tau)
                plsc.store_scatter(
                    gids, [off + iota], jnp.full((_L,), gid0 + g, jnp.int32))
                off = off + jnp.where(pc > 0, _L, 0)
            return pms[0], pms[1], pms[2], pms[3], tau_n, tau_nn, off

        pm0, pm1, pm2, pm3, tau, tau_n, off = lax.fori_loop(
            0, _STEPS // _UNROLL, p1,
            (pos_inf, pos_inf, pos_inf, pos_inf, pos_inf, pos_inf,
             jnp.zeros((_L,), jnp.int32)))
        ntrips = off[0] // _L

        def sel(t, carry):
            ts = list(carry)
            gid = gids[pl.ds(t * _L, _L)][0]
            base = gid * (_G * _L)
            for j in range(_G):
                c = rb[pl.ds(base + j * _L, _L)]
                for q in range(_K - 1):
                    n = jnp.minimum(ts[q], c)
                    c = jnp.maximum(ts[q], c)
                    ts[q] = n
                ts[_K - 1] = jnp.minimum(ts[_K - 1], c)
            return tuple(ts)

        tsel = lax.fori_loop(0, ntrips, sel, (pos_inf,) * _K)

        s = jnp.sort(tsel[0])
        for t in tsel[1:]:
            s = jnp.sort(jnp.minimum(s, jnp.flip(jnp.sort(t))))

        o_buf[pl.ds(r * _L, _L)] = s

    pltpu.sync_copy(o_buf, out_hbm.at[pl.ds(row0 * _L, _RPW * _L)])


_sc_call = pl.kernel(
    _sc_body,
    out_type=jax.ShapeDtypeStruct((_B * _L,), jnp.float32),
    mesh=plsc.VectorSubcoreMesh(core_axis_name="c", subcore_axis_name="s"),
    scratch_types=[
        pltpu.VMEM((_N,), jnp.float32),
        pltpu.VMEM((_N,), jnp.float32),
        pltpu.VMEM((_NGRP * _L,), jnp.int32),
        pltpu.VMEM((_RPW * _L,), jnp.float32),
        pltpu.SemaphoreType.DMA,
        pltpu.SemaphoreType.DMA,
    ],
    compiler_params=pltpu.CompilerParams(needs_layout_passes=False),
)

_COLS = 1024


def _tc_rowsum_body(x_ref, o_ref):
    i = pl.program_id(0)
    part = jnp.sum(jnp.exp(x_ref[...]), axis=1, keepdims=True)

    @pl.when(i == 0)
    def _init():
        o_ref[...] = part

    @pl.when(i > 0)
    def _acc():
        o_ref[...] += part


_tc_rowsum = pl.pallas_call(
    _tc_rowsum_body,
    grid=(_N // _COLS,),
    in_specs=[pl.BlockSpec((_B, _COLS), lambda i: (0, i))],
    out_specs=pl.BlockSpec((_B, 1), lambda i: (0, 0)),
    out_shape=jax.ShapeDtypeStruct((_B, 1), jnp.float32),
)


def _tc_comb_body(v_ref, s_ref, o_ref):
    sm = jnp.exp(v_ref[...]) / s_ref[...]
    t = _EPS - sm
    contrib = t * t * 0.5 - t
    keep = lax.broadcasted_iota(jnp.int32, (_B, _L), 1) < _K
    o_ref[0, 0] = jnp.sum(jnp.where(keep, contrib, 0.0)) * (1.0 / (_K * _B))


_tc_comb = pl.pallas_call(
    _tc_comb_body,
    out_shape=jax.ShapeDtypeStruct((1, 1), jnp.float32),
    out_specs=pl.BlockSpec(memory_space=pltpu.SMEM),
)


def kernel(x):
    v6 = _sc_call(x).reshape(_B, _L)
    s = _tc_rowsum(x)
    return _tc_comb(v6, s)[0, 0]
